# baseline (speedup 1.0000x reference)
import functools

import jax
import jax.numpy as jnp
from jax import lax
from jax.experimental import pallas as pl
from jax.experimental.pallas import tpu as pltpu
from jax.experimental.pallas import tpu_sc as plsc

NC = 2
NS = 16
NW = NC * NS
CHUNK = 96


def kernel(x, edge_index, edge_attr, batch, src_node_idx, dest_node_idx,
           W_emb, b_emb, W1, b1, W2, b2, bn3_gamma, bn3_beta,
           Wff, bff, Wff1, bff1, Wff2, bff2):
    n, d = x.shape
    h = W1.shape[1]
    c_out = Wff2.shape[1]
    e = edge_index.shape[1]
    nb = src_node_idx.shape[0]
    n_per = n // nb

    ept = -(-e // (NW * 3 * CHUNK)) * (3 * CHUNK)
    nchunks = ept // CHUNK
    e_pad = ept * NW
    S = -(-(n + 1) // (NS * 16)) * 16
    R = S * NS

    NG = nchunks
    if NG > 24:
        for cand in range(24, 2, -3):
            if nchunks % cand == 0 and cand % 3 == 0:
                NG = cand
                break
    nst = nchunks // NG

    row = edge_index[0]
    col = edge_index[1]
    pad = e_pad - e
    pad_r = (jnp.arange(pad, dtype=row.dtype) * 7) % n
    pad_c = n + (jnp.arange(pad, dtype=col.dtype) % (R - n))
    r_pad = jnp.concatenate([row, pad_r]).reshape(NW, nst, NG, CHUNK)
    c_pad = jnp.concatenate([col, pad_c]).reshape(NW, nst, NG, CHUNK)

    mesh = plsc.VectorSubcoreMesh(core_axis_name="c", subcore_axis_name="s",
                                  num_cores=NC, num_subcores=NS)

    @functools.partial(
        pl.kernel,
        out_type=jax.ShapeDtypeStruct((NC, R), jnp.float32),
        mesh=mesh,
        scratch_types=[
            pltpu.VMEM((nst, NG, CHUNK), jnp.int32),
            pltpu.VMEM((CHUNK,), jnp.float32),
            pltpu.VMEM_SHARED((R,), jnp.float32),
            pltpu.SemaphoreType.DMA,
        ],
    )
    def deg_kernel(c_hbm, out_hbm, c_v, ones_v, acc, semd):
        cid = lax.axis_index("c")
        sid = lax.axis_index("s")
        blk = cid * NS + sid
        for i in range(CHUNK // 16):
            ones_v[pl.ds(i * 16, 16)] = jnp.zeros((16,), jnp.float32)
        znf, znr = divmod(S, CHUNK)
        for k in range(znf):
            pltpu.sync_copy(ones_v, acc.at[pl.ds(sid * S + k * CHUNK, CHUNK)])
        if znr:
            pltpu.sync_copy(ones_v.at[pl.ds(0, znr)],
                            acc.at[pl.ds(sid * S + znf * CHUNK, znr)])
        for i in range(CHUNK // 16):
            ones_v[pl.ds(i * 16, 16)] = jnp.full((16,), 1.0, jnp.float32)
        pltpu.sync_copy(c_hbm.at[blk], c_v)
        plsc.subcore_barrier()

        LAG = 8
        for st in range(nst):
            def body(j, carry):
                pltpu.async_copy(ones_v, acc.at[c_v.at[st, j]], semd, add=True)
                jg = st * NG + j

                @pl.when(jg >= LAG)
                def _():
                    pltpu.make_async_copy(
                        ones_v, acc.at[c_v.at[st, j]], semd).wait()
                return carry

            lax.fori_loop(0, NG, body, 0)
        for k in range(LAG):
            pltpu.make_async_copy(ones_v, acc.at[c_v.at[0, 0]], semd).wait()
        plsc.subcore_barrier()
        pltpu.sync_copy(acc.at[pl.ds(sid * S, S)], out_hbm.at[cid, pl.ds(sid * S, S)])

    @functools.partial(
        pl.kernel,
        out_type=jax.ShapeDtypeStruct((NC, R, h), jnp.float32),
        mesh=mesh,
        scratch_types=[
            pltpu.VMEM((NG, CHUNK), jnp.int32),
            pltpu.VMEM((NG, CHUNK), jnp.int32),
            pltpu.VMEM((CHUNK, h), jnp.float32),
            pltpu.VMEM((CHUNK, h), jnp.float32),
            pltpu.VMEM((CHUNK, h), jnp.float32),
            pltpu.VMEM_SHARED((R, h), jnp.float32),
            pltpu.SemaphoreType.DMA,
            pltpu.SemaphoreType.DMA,
            pltpu.SemaphoreType.DMA,
            pltpu.SemaphoreType.DMA,
            pltpu.SemaphoreType.DMA,
            pltpu.SemaphoreType.DMA,
        ],
    )
    def agg_kernel(g_hbm, r_hbm, c_hbm, out_hbm,
                   r_v, c_v, buf0, buf1, buf2, acc,
                   semg0, semg1, semg2, sems0, sems1, sems2):
        cid = lax.axis_index("c")
        sid = lax.axis_index("s")
        blk = cid * NS + sid
        bufs = (buf0, buf1, buf2)
        semg = (semg0, semg1, semg2)
        sems = (sems0, sems1, sems2)
        zv = jnp.zeros((16,), jnp.float32)

        def zrow_body(i, carry):
            for q in range(h // 16):
                buf0[i, pl.ds(q * 16, 16)] = zv
            return carry

        lax.fori_loop(0, CHUNK, zrow_body, 0)
        nfull, rem = divmod(S, CHUNK)
        for k in range(nfull):
            pltpu.sync_copy(buf0, acc.at[pl.ds(sid * S + k * CHUNK, CHUNK)])
        if rem:
            pltpu.sync_copy(buf0.at[pl.ds(0, rem)],
                            acc.at[pl.ds(sid * S + nfull * CHUNK, rem)])
        pltpu.sync_copy(r_hbm.at[blk, 0], r_v)
        pltpu.sync_copy(c_hbm.at[blk, 0], c_v)
        plsc.subcore_barrier()

        ng3 = NG // 3
        for st in range(nst):
            if st > 0:
                pltpu.sync_copy(r_hbm.at[blk, st], r_v)
                pltpu.sync_copy(c_hbm.at[blk, st], c_v)
            pltpu.async_copy(g_hbm.at[r_v.at[0]], buf0, semg0)
            pltpu.async_copy(g_hbm.at[r_v.at[1]], buf1, semg1)

            def body(jj, carry):
                for k in range(3):
                    j = 3 * jj + k
                    b = k
                    b2 = (k + 2) % 3
                    pltpu.make_async_copy(g_hbm.at[r_v.at[j]], bufs[b], semg[b]).wait()
                    pltpu.async_copy(bufs[b], acc.at[c_v.at[j]], sems[b], add=True)
                    if k == 0:
                        @pl.when(jj > 0)
                        def _():
                            pltpu.make_async_copy(
                                bufs[b2], acc.at[c_v.at[j]], sems[b2]).wait()
                    else:
                        pltpu.make_async_copy(
                            bufs[b2], acc.at[c_v.at[j]], sems[b2]).wait()

                    @pl.when(j + 2 < NG)
                    def _():
                        pltpu.async_copy(g_hbm.at[r_v.at[j + 2]], bufs[b2], semg[b2])
                return carry

            lax.fori_loop(0, ng3, body, 0)
            pltpu.make_async_copy(bufs[2], acc.at[c_v.at[NG - 1]], sems[2]).wait()
        plsc.subcore_barrier()
        pltpu.sync_copy(acc.at[pl.ds(sid * S, S)], out_hbm.at[cid, pl.ds(sid * S, S)])

    deg2 = deg_kernel(c_pad)

    BN = 2000
    nblk = n // BN
    full = lambda *s: pl.BlockSpec(s, lambda i: (0,) * len(s))

    def t1a_body(x_ref, we_ref, be_ref, w1_ref, o_ref):
        h0 = jnp.dot(x_ref[...], we_ref[...], preferred_element_type=jnp.float32) + be_ref[...]
        o_ref[...] = jnp.dot(h0, w1_ref[...], preferred_element_type=jnp.float32)

    m1 = pl.pallas_call(
        t1a_body, out_shape=jax.ShapeDtypeStruct((n, h), jnp.float32),
        grid=(nblk,),
        in_specs=[pl.BlockSpec((BN, d), lambda i: (i, 0)),
                  full(d, h), full(1, h), full(h, h)],
        out_specs=pl.BlockSpec((BN, h), lambda i: (i, 0)),
    )(x, W_emb, b_emb.reshape(1, h), W1)

    def t1b_body(d0_ref, d1_ref, m_ref, g_ref, dinv_ref):
        dv = lax.rsqrt(d0_ref[...] + d1_ref[...] + 1.0)
        dinv_ref[...] = dv
        g_ref[...] = dv * m_ref[...]

    g1, dinv_c = pl.pallas_call(
        t1b_body,
        out_shape=(jax.ShapeDtypeStruct((n, h), jnp.float32),
                   jax.ShapeDtypeStruct((n, 1), jnp.float32)),
        grid=(nblk,),
        in_specs=[pl.BlockSpec((BN, 1), lambda i: (i, 0)),
                  pl.BlockSpec((BN, 1), lambda i: (i, 0)),
                  pl.BlockSpec((BN, h), lambda i: (i, 0))],
        out_specs=(pl.BlockSpec((BN, h), lambda i: (i, 0)),
                   pl.BlockSpec((BN, 1), lambda i: (i, 0))),
    )(deg2[0, :n, None], deg2[1, :n, None], m1)

    a1 = agg_kernel(g1, r_pad, c_pad)

    def t2_body(a_ref, g_ref, dinv_ref, b1_ref, w2_ref, o_ref):
        av = a_ref[0] + a_ref[1]
        h1 = jnp.maximum(
            dinv_ref[...] * (av + g_ref[...]) + b1_ref[...], 0.0)
        o_ref[...] = dinv_ref[...] * jnp.dot(h1, w2_ref[...], preferred_element_type=jnp.float32)

    g2 = pl.pallas_call(
        t2_body, out_shape=jax.ShapeDtypeStruct((n, h), jnp.float32),
        grid=(nblk,),
        in_specs=[pl.BlockSpec((2, BN, h), lambda i: (0, i, 0)),
                  pl.BlockSpec((BN, h), lambda i: (i, 0)),
                  pl.BlockSpec((BN, 1), lambda i: (i, 0)),
                  full(1, h), full(h, h)],
        out_specs=pl.BlockSpec((BN, h), lambda i: (i, 0)),
    )(a1, g1, dinv_c, b1.reshape(1, h), W2)

    a2 = agg_kernel(g2, r_pad, c_pad)

    def t3_body(a_ref, g_ref, dinv_ref, b2_ref, si_ref, di_ref,
                wff_ref, bff_ref, bng_ref, bnb_ref, wf1_ref, bf1_ref,
                wf2_ref, bf2_ref, o_ref, h2_ref, sr_ref, dr_ref):
        av = a_ref[0, :n, :] + a_ref[1, :n, :]
        h2 = dinv_ref[...] * (av + g_ref[...]) + b2_ref[...]
        h2_ref[...] = h2
        for bb in range(nb):
            i_s = si_ref[bb] + bb * n_per
            i_d = di_ref[bb] + bb * n_per
            sr_ref[pl.ds(bb, 1), :] = h2_ref[pl.ds(i_s, 1), :]
            dr_ref[pl.ds(bb, 1), :] = h2_ref[pl.ds(i_d, 1), :]
        sd = (jnp.dot(sr_ref[...], wff_ref[h:2 * h, :], preferred_element_type=jnp.float32)
              + jnp.dot(dr_ref[...], wff_ref[2 * h:3 * h, :], preferred_element_type=jnp.float32))
        rid = lax.broadcasted_iota(jnp.int32, (n, nb), 0) // n_per
        cix = lax.broadcasted_iota(jnp.int32, (n, nb), 1)
        emat = (rid == cix).astype(jnp.float32)
        u = (jnp.dot(h2, wff_ref[0:h, :], preferred_element_type=jnp.float32)
             + jnp.dot(emat, sd, preferred_element_type=jnp.float32) + bff_ref[...])
        mu = jnp.mean(u, axis=0, keepdims=True)
        var = jnp.mean((u - mu) * (u - mu), axis=0, keepdims=True)
        z = jnp.maximum((u - mu) * lax.rsqrt(var + 1e-5) * bng_ref[...] + bnb_ref[...], 0.0)
        y = jnp.maximum(
            jnp.dot(z, wf1_ref[...], preferred_element_type=jnp.float32) + bf1_ref[...], 0.0)
        o = jnp.dot(y, wf2_ref[...], preferred_element_type=jnp.float32) + bf2_ref[...]
        m = jnp.max(o, axis=1, keepdims=True)
        lse = jnp.log(jnp.sum(jnp.exp(o - m), axis=1, keepdims=True)) + m
        o_ref[...] = o - lse

    vspec = pl.BlockSpec(memory_space=pltpu.VMEM)
    sspec = pl.BlockSpec(memory_space=pltpu.SMEM)
    out = pl.pallas_call(
        t3_body,
        out_shape=jax.ShapeDtypeStruct((n, c_out), jnp.float32),
        in_specs=[vspec, vspec, vspec, vspec, sspec, sspec,
                  vspec, vspec, vspec, vspec, vspec, vspec, vspec, vspec],
        scratch_shapes=[pltpu.VMEM((n, h), jnp.float32),
                        pltpu.VMEM((nb, h), jnp.float32),
                        pltpu.VMEM((nb, h), jnp.float32)],
    )(a2, g2, dinv_c, b2.reshape(1, h),
      src_node_idx, dest_node_idx, Wff, bff.reshape(1, h),
      bn3_gamma.reshape(1, h), bn3_beta.reshape(1, h),
      Wff1, bff1.reshape(1, h), Wff2, bff2.reshape(1, c_out))
    return out


# speedup vs baseline: 1.0063x; 1.0063x over previous
import functools

import jax
import jax.numpy as jnp
from jax import lax
from jax.experimental import pallas as pl
from jax.experimental.pallas import tpu as pltpu
from jax.experimental.pallas import tpu_sc as plsc

NC = 2
NS = 16
NW = NC * NS
CHUNK = 96


def kernel(x, edge_index, edge_attr, batch, src_node_idx, dest_node_idx,
           W_emb, b_emb, W1, b1, W2, b2, bn3_gamma, bn3_beta,
           Wff, bff, Wff1, bff1, Wff2, bff2):
    n, d = x.shape
    h = W1.shape[1]
    c_out = Wff2.shape[1]
    e = edge_index.shape[1]
    nb = src_node_idx.shape[0]
    n_per = n // nb

    ept = -(-e // (NW * 3 * CHUNK)) * (3 * CHUNK)
    nchunks = ept // CHUNK
    e_pad = ept * NW
    S = -(-(n + 1) // (NS * 16)) * 16
    R = S * NS

    NG = nchunks
    if NG > 24:
        for cand in range(24, 2, -3):
            if nchunks % cand == 0 and cand % 3 == 0:
                NG = cand
                break
    nst = nchunks // NG

    row = edge_index[0]
    col = edge_index[1]
    pad = e_pad - e
    pad_r = (jnp.arange(pad, dtype=row.dtype) * 7) % n
    pad_c = n + (jnp.arange(pad, dtype=col.dtype) % (R - n))
    r_pad = jnp.concatenate([row, pad_r]).reshape(NW, nst, NG, CHUNK)
    c_pad = jnp.concatenate([col, pad_c]).reshape(NW, nst, NG, CHUNK)

    mesh = plsc.VectorSubcoreMesh(core_axis_name="c", subcore_axis_name="s",
                                  num_cores=NC, num_subcores=NS)

    @functools.partial(
        pl.kernel,
        out_type=jax.ShapeDtypeStruct((NC, R), jnp.float32),
        mesh=mesh,
        scratch_types=[
            pltpu.VMEM((nst, NG, CHUNK), jnp.int32),
            pltpu.VMEM((CHUNK,), jnp.float32),
            pltpu.VMEM_SHARED((R,), jnp.float32),
            pltpu.SemaphoreType.DMA,
        ],
    )
    def deg_kernel(c_hbm, out_hbm, c_v, ones_v, acc, semd):
        cid = lax.axis_index("c")
        sid = lax.axis_index("s")
        blk = cid * NS + sid
        for i in range(CHUNK // 16):
            ones_v[pl.ds(i * 16, 16)] = jnp.zeros((16,), jnp.float32)
        znf, znr = divmod(S, CHUNK)
        for k in range(znf):
            pltpu.sync_copy(ones_v, acc.at[pl.ds(sid * S + k * CHUNK, CHUNK)])
        if znr:
            pltpu.sync_copy(ones_v.at[pl.ds(0, znr)],
                            acc.at[pl.ds(sid * S + znf * CHUNK, znr)])
        for i in range(CHUNK // 16):
            ones_v[pl.ds(i * 16, 16)] = jnp.full((16,), 1.0, jnp.float32)
        pltpu.sync_copy(c_hbm.at[blk], c_v)
        plsc.subcore_barrier()

        LAG = 8
        for st in range(nst):
            def body(j, carry):
                pltpu.async_copy(ones_v, acc.at[c_v.at[st, j]], semd, add=True)
                jg = st * NG + j

                @pl.when(jg >= LAG)
                def _():
                    pltpu.make_async_copy(
                        ones_v, acc.at[c_v.at[st, j]], semd).wait()
                return carry

            lax.fori_loop(0, NG, body, 0)
        for k in range(LAG):
            pltpu.make_async_copy(ones_v, acc.at[c_v.at[0, 0]], semd).wait()
        plsc.subcore_barrier()
        pltpu.sync_copy(acc.at[pl.ds(sid * S, S)], out_hbm.at[cid, pl.ds(sid * S, S)])

    @functools.partial(
        pl.kernel,
        out_type=jax.ShapeDtypeStruct((NC, R, h), jnp.float32),
        mesh=mesh,
        scratch_types=[
            pltpu.VMEM((NG, CHUNK), jnp.int32),
            pltpu.VMEM((NG, CHUNK), jnp.int32),
            pltpu.VMEM((CHUNK, h), jnp.float32),
            pltpu.VMEM((CHUNK, h), jnp.float32),
            pltpu.VMEM((CHUNK, h), jnp.float32),
            pltpu.VMEM_SHARED((R, h), jnp.float32),
            pltpu.SemaphoreType.DMA,
            pltpu.SemaphoreType.DMA,
            pltpu.SemaphoreType.DMA,
            pltpu.SemaphoreType.DMA,
            pltpu.SemaphoreType.DMA,
            pltpu.SemaphoreType.DMA,
        ],
    )
    def agg_kernel(g_hbm, r_hbm, c_hbm, out_hbm,
                   r_v, c_v, buf0, buf1, buf2, acc,
                   semg0, semg1, semg2, sems0, sems1, sems2):
        cid = lax.axis_index("c")
        sid = lax.axis_index("s")
        blk = cid * NS + sid
        bufs = (buf0, buf1, buf2)
        semg = (semg0, semg1, semg2)
        sems = (sems0, sems1, sems2)
        zv = jnp.zeros((16,), jnp.float32)

        def zrow_body(i, carry):
            for q in range(h // 16):
                buf0[i, pl.ds(q * 16, 16)] = zv
            return carry

        lax.fori_loop(0, CHUNK, zrow_body, 0)
        nfull, rem = divmod(S, CHUNK)
        for k in range(nfull):
            pltpu.sync_copy(buf0, acc.at[pl.ds(sid * S + k * CHUNK, CHUNK)])
        if rem:
            pltpu.sync_copy(buf0.at[pl.ds(0, rem)],
                            acc.at[pl.ds(sid * S + nfull * CHUNK, rem)])
        pltpu.sync_copy(r_hbm.at[blk, 0], r_v)
        pltpu.sync_copy(c_hbm.at[blk, 0], c_v)
        plsc.subcore_barrier()

        ng3 = NG // 3
        for st in range(nst):
            if st > 0:
                pltpu.sync_copy(r_hbm.at[blk, st], r_v)
                pltpu.sync_copy(c_hbm.at[blk, st], c_v)
            pltpu.async_copy(g_hbm.at[r_v.at[0]], buf0, semg0)
            pltpu.async_copy(g_hbm.at[r_v.at[1]], buf1, semg1)

            def body(jj, carry):
                for k in range(3):
                    j = 3 * jj + k
                    b = k
                    b2 = (k + 2) % 3
                    pltpu.make_async_copy(g_hbm.at[r_v.at[j]], bufs[b], semg[b]).wait()
                    pltpu.async_copy(bufs[b], acc.at[c_v.at[j]], sems[b], add=True)
                    if k == 0:
                        @pl.when(jj > 0)
                        def _():
                            pltpu.make_async_copy(
                                bufs[b2], acc.at[c_v.at[j]], sems[b2]).wait()
                    else:
                        pltpu.make_async_copy(
                            bufs[b2], acc.at[c_v.at[j]], sems[b2]).wait()

                    @pl.when(j + 2 < NG)
                    def _():
                        pltpu.async_copy(g_hbm.at[r_v.at[j + 2]], bufs[b2], semg[b2])
                return carry

            lax.fori_loop(0, ng3, body, 0)
            pltpu.make_async_copy(bufs[2], acc.at[c_v.at[NG - 1]], sems[2]).wait()
        plsc.subcore_barrier()
        pltpu.sync_copy(acc.at[pl.ds(sid * S, S)], out_hbm.at[cid, pl.ds(sid * S, S)])

    deg2 = deg_kernel(c_pad)

    BN = 2000
    nblk = n // BN
    full = lambda *s: pl.BlockSpec(s, lambda i: (0,) * len(s))

    def t1_body(x_ref, we_ref, be_ref, w1_ref, d0_ref, d1_ref, g_ref, dinv_ref):
        dv = lax.rsqrt(d0_ref[...] + d1_ref[...] + 1.0)
        dinv_ref[...] = dv
        h0 = jnp.dot(x_ref[...], we_ref[...], preferred_element_type=jnp.float32) + be_ref[...]
        g_ref[...] = dv * jnp.dot(h0, w1_ref[...], preferred_element_type=jnp.float32)

    g1, dinv_c = pl.pallas_call(
        t1_body,
        out_shape=(jax.ShapeDtypeStruct((n, h), jnp.float32),
                   jax.ShapeDtypeStruct((n, 1), jnp.float32)),
        grid=(nblk,),
        in_specs=[pl.BlockSpec((BN, d), lambda i: (i, 0)),
                  full(d, h), full(1, h), full(h, h),
                  pl.BlockSpec((BN, 1), lambda i: (i, 0)),
                  pl.BlockSpec((BN, 1), lambda i: (i, 0))],
        out_specs=(pl.BlockSpec((BN, h), lambda i: (i, 0)),
                   pl.BlockSpec((BN, 1), lambda i: (i, 0))),
    )(x, W_emb, b_emb.reshape(1, h), W1, deg2[0, :n, None], deg2[1, :n, None])

    a1 = agg_kernel(g1, r_pad, c_pad)

    def t2_body(a_ref, g_ref, dinv_ref, b1_ref, w2_ref, o_ref):
        av = a_ref[0] + a_ref[1]
        h1 = jnp.maximum(
            dinv_ref[...] * (av + g_ref[...]) + b1_ref[...], 0.0)
        o_ref[...] = dinv_ref[...] * jnp.dot(h1, w2_ref[...], preferred_element_type=jnp.float32)

    g2 = pl.pallas_call(
        t2_body, out_shape=jax.ShapeDtypeStruct((n, h), jnp.float32),
        grid=(nblk,),
        in_specs=[pl.BlockSpec((2, BN, h), lambda i: (0, i, 0)),
                  pl.BlockSpec((BN, h), lambda i: (i, 0)),
                  pl.BlockSpec((BN, 1), lambda i: (i, 0)),
                  full(1, h), full(h, h)],
        out_specs=pl.BlockSpec((BN, h), lambda i: (i, 0)),
    )(a1, g1, dinv_c, b1.reshape(1, h), W2)

    a2 = agg_kernel(g2, r_pad, c_pad)

    def t3_body(a_ref, g_ref, dinv_ref, b2_ref, si_ref, di_ref,
                wff_ref, bff_ref, bng_ref, bnb_ref, wf1_ref, bf1_ref,
                wf2_ref, bf2_ref, o_ref, h2_ref, sr_ref, dr_ref):
        av = a_ref[0, :n, :] + a_ref[1, :n, :]
        h2 = dinv_ref[...] * (av + g_ref[...]) + b2_ref[...]
        h2_ref[...] = h2
        for bb in range(nb):
            i_s = si_ref[bb] + bb * n_per
            i_d = di_ref[bb] + bb * n_per
            sr_ref[pl.ds(bb, 1), :] = h2_ref[pl.ds(i_s, 1), :]
            dr_ref[pl.ds(bb, 1), :] = h2_ref[pl.ds(i_d, 1), :]
        sd = (jnp.dot(sr_ref[...], wff_ref[h:2 * h, :], preferred_element_type=jnp.float32)
              + jnp.dot(dr_ref[...], wff_ref[2 * h:3 * h, :], preferred_element_type=jnp.float32))
        rid = lax.broadcasted_iota(jnp.int32, (n, nb), 0) // n_per
        cix = lax.broadcasted_iota(jnp.int32, (n, nb), 1)
        emat = (rid == cix).astype(jnp.float32)
        u = (jnp.dot(h2, wff_ref[0:h, :], preferred_element_type=jnp.float32)
             + jnp.dot(emat, sd, preferred_element_type=jnp.float32) + bff_ref[...])
        mu = jnp.mean(u, axis=0, keepdims=True)
        var = jnp.mean((u - mu) * (u - mu), axis=0, keepdims=True)
        z = jnp.maximum((u - mu) * lax.rsqrt(var + 1e-5) * bng_ref[...] + bnb_ref[...], 0.0)
        y = jnp.maximum(
            jnp.dot(z, wf1_ref[...], preferred_element_type=jnp.float32) + bf1_ref[...], 0.0)
        o = jnp.dot(y, wf2_ref[...], preferred_element_type=jnp.float32) + bf2_ref[...]
        m = jnp.max(o, axis=1, keepdims=True)
        lse = jnp.log(jnp.sum(jnp.exp(o - m), axis=1, keepdims=True)) + m
        o_ref[...] = o - lse

    vspec = pl.BlockSpec(memory_space=pltpu.VMEM)
    sspec = pl.BlockSpec(memory_space=pltpu.SMEM)
    out = pl.pallas_call(
        t3_body,
        out_shape=jax.ShapeDtypeStruct((n, c_out), jnp.float32),
        in_specs=[vspec, vspec, vspec, vspec, sspec, sspec,
                  vspec, vspec, vspec, vspec, vspec, vspec, vspec, vspec],
        scratch_shapes=[pltpu.VMEM((n, h), jnp.float32),
                        pltpu.VMEM((nb, h), jnp.float32),
                        pltpu.VMEM((nb, h), jnp.float32)],
    )(a2, g2, dinv_c, b2.reshape(1, h),
      src_node_idx, dest_node_idx, Wff, bff.reshape(1, h),
      bn3_gamma.reshape(1, h), bn3_beta.reshape(1, h),
      Wff1, bff1.reshape(1, h), Wff2, bff2.reshape(1, c_out))
    return out
